# Initial kernel scaffold; baseline (speedup 1.0000x reference)
import jax
import jax.numpy as jnp
from jax import lax
from jax.experimental import pallas as pl
from jax.experimental.pallas import tpu as pltpu
from jax.experimental.pallas import tpu_sc as plsc

N_NODES = 10000
N_EDGES = 160000
IN_FEATS = 256

_NUM_WORKERS = 32
_CHUNK = N_EDGES // _NUM_WORKERS
_LANES = 16

_BE = 2000


def _sc_same_kernel(h_hbm, src_hbm, dst_hbm, out_hbm, h_v, src_v, dst_v,
                    same_v, sem):
    wid = lax.axis_index("s") * 2 + lax.axis_index("c")
    base = wid * _CHUNK
    copy_h = pltpu.async_copy(h_hbm, h_v, sem)
    pltpu.sync_copy(src_hbm.at[pl.ds(base, _CHUNK)], src_v)
    pltpu.sync_copy(dst_hbm.at[pl.ds(base, _CHUNK)], dst_v)
    copy_h.wait()

    @pl.loop(0, _CHUNK, step=_LANES)
    def _(i):
        idx_s = src_v[pl.ds(i, _LANES)]
        idx_d = dst_v[pl.ds(i, _LANES)]
        vs = plsc.load_gather(h_v, [idx_s])
        vd = plsc.load_gather(h_v, [idx_d])
        same_v[pl.ds(i, _LANES)] = jnp.where(
            vs == vd, jnp.float32(1.0), jnp.float32(0.0))

    pltpu.sync_copy(same_v, out_hbm.at[pl.ds(base, _CHUNK)])


@jax.jit
def _sc_same(h32, src, dst):
    mesh = plsc.VectorSubcoreMesh(core_axis_name="c", subcore_axis_name="s")
    kern = pl.kernel(
        _sc_same_kernel,
        mesh=mesh,
        out_type=jax.ShapeDtypeStruct((N_EDGES,), jnp.float32),
        scratch_types=[
            pltpu.VMEM((N_NODES,), jnp.int32),
            pltpu.VMEM((_CHUNK,), jnp.int32),
            pltpu.VMEM((_CHUNK,), jnp.int32),
            pltpu.VMEM((_CHUNK,), jnp.float32),
            pltpu.SemaphoreType.DMA,
        ],
    )
    return kern(h32, src, dst)


def _tc_expand_kernel(same_ref, emb_ref, out_ref):
    s = same_ref[...]
    w0 = emb_ref[0:1, :]
    w1 = emb_ref[1:2, :]
    out_ref[...] = w0 + s * (w1 - w0)


@jax.jit
def _tc_expand(same2d, emb_weight):
    return pl.pallas_call(
        _tc_expand_kernel,
        grid=(N_EDGES // _BE,),
        in_specs=[
            pl.BlockSpec((_BE, 1), lambda i: (i, 0)),
            pl.BlockSpec((2, IN_FEATS), lambda i: (0, 0)),
        ],
        out_specs=pl.BlockSpec((_BE, IN_FEATS), lambda i: (i, 0)),
        out_shape=jax.ShapeDtypeStruct((N_EDGES, IN_FEATS), jnp.float32),
    )(same2d, emb_weight)


def kernel(h, edge_index, emb_weight):
    h32 = h.astype(jnp.int32)
    src = edge_index[0].astype(jnp.int32)
    dst = edge_index[1].astype(jnp.int32)
    same = _sc_same(h32, src, dst)
    return _tc_expand(same.reshape(N_EDGES, 1), emb_weight)


# speedup vs baseline: 12.7314x; 12.7314x over previous
import dataclasses

import jax
import jax.numpy as jnp
from jax import lax
from jax.experimental import pallas as pl
from jax.experimental.pallas import tpu as pltpu
from jax.experimental.pallas import tpu_sc as plsc

N_NODES = 10000
N_EDGES = 160000
IN_FEATS = 256

_NUM_WORKERS = 32
_LANES = 16
_CHUNK = 5008
_E_PAD = _NUM_WORKERS * _CHUNK

_BE = 2000


def _sc_same_kernel(h_hbm, src_hbm, dst_hbm, out_hbm, h_v, src_v, dst_v,
                    same_v, sem):
    wid = lax.axis_index("s") * 2 + lax.axis_index("c")
    base = wid * _CHUNK
    copy_h = pltpu.async_copy(h_hbm, h_v, sem)
    pltpu.sync_copy(src_hbm.at[pl.ds(base, _CHUNK)], src_v)
    pltpu.sync_copy(dst_hbm.at[pl.ds(base, _CHUNK)], dst_v)
    copy_h.wait()

    @pl.loop(0, _CHUNK, step=_LANES)
    def _(i):
        idx_s = src_v[pl.ds(i, _LANES)]
        idx_d = dst_v[pl.ds(i, _LANES)]
        vs = plsc.load_gather(h_v, [idx_s])
        vd = plsc.load_gather(h_v, [idx_d])
        same_v[pl.ds(i, _LANES)] = jnp.where(
            vs == vd, jnp.float32(1.0), jnp.float32(0.0))

    pltpu.sync_copy(same_v, out_hbm.at[pl.ds(base, _CHUNK)])


@jax.jit
def _sc_same(h32, src, dst):
    mesh = plsc.VectorSubcoreMesh(core_axis_name="c", subcore_axis_name="s")
    cp = pltpu.CompilerParams()
    if "needs_layout_passes" in pltpu.CompilerParams.__dataclass_fields__:
        cp = dataclasses.replace(cp, needs_layout_passes=False)
    kern = pl.kernel(
        _sc_same_kernel,
        mesh=mesh,
        compiler_params=cp,
        out_type=jax.ShapeDtypeStruct((_E_PAD,), jnp.float32),
        scratch_types=[
            pltpu.VMEM((N_NODES,), jnp.int32),
            pltpu.VMEM((_CHUNK,), jnp.int32),
            pltpu.VMEM((_CHUNK,), jnp.int32),
            pltpu.VMEM((_CHUNK,), jnp.float32),
            pltpu.SemaphoreType.DMA,
        ],
    )
    return kern(h32, src, dst)


def _tc_expand_kernel(same_ref, emb_ref, out_ref):
    s = same_ref[...]
    w0 = emb_ref[0:1, :]
    w1 = emb_ref[1:2, :]
    out_ref[...] = w0 + s * (w1 - w0)


@jax.jit
def _tc_expand(same2d, emb_weight):
    return pl.pallas_call(
        _tc_expand_kernel,
        grid=(N_EDGES // _BE,),
        in_specs=[
            pl.BlockSpec((_BE, 1), lambda i: (i, 0)),
            pl.BlockSpec((2, IN_FEATS), lambda i: (0, 0)),
        ],
        out_specs=pl.BlockSpec((_BE, IN_FEATS), lambda i: (i, 0)),
        out_shape=jax.ShapeDtypeStruct((N_EDGES, IN_FEATS), jnp.float32),
    )(same2d, emb_weight)


def kernel(h, edge_index, emb_weight):
    h32 = h.astype(jnp.int32)
    ei = edge_index.astype(jnp.int32)
    pad = jnp.zeros((2, _E_PAD - N_EDGES), jnp.int32)
    ei = jnp.concatenate([ei, pad], axis=1)
    same = _sc_same(h32, ei[0], ei[1])
    return _tc_expand(same[:N_EDGES].reshape(N_EDGES, 1), emb_weight)
